# baseline (speedup 1.0000x reference)
import jax
import jax.numpy as jnp
from jax.experimental import pallas as pl
from jax.experimental.pallas import tpu as pltpu
from jax.experimental.pallas import tpu_sc as plsc

_B, _S, _P, _D, _H, _V, _FF = 2, 2048, 20, 1024, 16, 8192, 4096
_DH = _D // _H
_T = _P + _S
_TP = 2176
_N = _B * _TP
_RB = 256
_RQ = 128
_EPS = 1e-5
_NEG = -1e30


def _ln(x):
    mu = jnp.mean(x, axis=-1, keepdims=True)
    var = jnp.mean(jnp.square(x - mu), axis=-1, keepdims=True)
    return (x - mu) * jax.lax.rsqrt(var + _EPS)


def _sc_gather(emb, ids_flat):
    n = ids_flat.shape[0]
    v, d = emb.shape
    c = 4
    dc = d // c
    w = 128
    nc = n * c
    idx = (ids_flat[:, None] * c
           + jnp.arange(c, dtype=jnp.int32)[None, :]).reshape(1, nc)
    embc = emb.reshape(v * c, dc)
    mesh = plsc.VectorSubcoreMesh(core_axis_name="c", subcore_axis_name="s")

    @pl.kernel(out_type=jax.ShapeDtypeStruct((nc, dc), emb.dtype), mesh=mesh)
    def k(emb_hbm, i_hbm, o_hbm):
        def body(i_vmem, o_vmem):
            pltpu.sync_copy(emb_hbm.at[i_vmem.at[0]], o_vmem)

        pltpu.emit_pipeline(
            body,
            grid=(nc // w,),
            in_specs=[pl.BlockSpec((1, w), index_map=lambda i: (0, i))],
            out_specs=[pl.BlockSpec((w, dc), index_map=lambda i: (i, 0))],
            core_axis_name=("c", "s"),
            dimension_semantics=(pltpu.PARALLEL,),
        )(i_hbm, o_hbm)

    return k(embc, idx).reshape(n, d)


def _qkv_body(x_ref, wq_ref, wk_ref, wv_ref, q_ref, k_ref, v_ref):
    h = _ln(x_ref[...]).astype(jnp.bfloat16)
    q_ref[...] = jnp.dot(h, wq_ref[...],
                         preferred_element_type=jnp.float32).astype(jnp.bfloat16)
    k_ref[...] = jnp.dot(h, wk_ref[...],
                         preferred_element_type=jnp.float32).astype(jnp.bfloat16)
    v_ref[...] = jnp.dot(h, wv_ref[...],
                         preferred_element_type=jnp.float32).astype(jnp.bfloat16)


def _qkv(x, wq, wk, wv):
    row = pl.BlockSpec((_RB, _D), lambda i: (i, 0))
    wsp = pl.BlockSpec((_D, _D), lambda i: (0, 0))
    out = jax.ShapeDtypeStruct((_N, _D), jnp.bfloat16)
    return pl.pallas_call(
        _qkv_body,
        grid=(_N // _RB,),
        in_specs=[row, wsp, wsp, wsp],
        out_specs=[row, row, row],
        out_shape=[out, out, out],
    )(x, wq, wk, wv)


def _attn_one(q, k, v):
    s = jax.lax.dot_general(q, k, (((1,), (1,)), ((), ())),
                            preferred_element_type=jnp.float32)
    s = s * (1.0 / 8.0)
    col = jax.lax.broadcasted_iota(jnp.int32, s.shape, 1)
    s = jnp.where(col < _T, s, _NEG)
    m = jnp.max(s, axis=-1, keepdims=True)
    e = jnp.exp(s - m)
    p = (e / jnp.sum(e, axis=-1, keepdims=True)).astype(jnp.bfloat16)
    return jnp.dot(p, v, preferred_element_type=jnp.float32)


def _attn_body(q_ref, k_hbm, v_hbm, o_ref, kbuf, vbuf, sem_k, sem_v):
    b = pl.program_id(0)
    hp = pl.program_id(1)
    qi = pl.program_id(2)
    ck = pltpu.make_async_copy(
        k_hbm.at[b, :, pl.ds(hp * 2 * _DH, 2 * _DH)], kbuf, sem_k)
    cv = pltpu.make_async_copy(
        v_hbm.at[b, :, pl.ds(hp * 2 * _DH, 2 * _DH)], vbuf, sem_v)

    @pl.when(qi == 0)
    def _():
        ck.start()
        cv.start()
        ck.wait()
        cv.wait()

    q = q_ref[0]
    k = kbuf[...]
    v = vbuf[...]
    ca = _attn_one(q[:, :_DH], k[:, :_DH], v[:, :_DH])
    cb = _attn_one(q[:, _DH:], k[:, _DH:], v[:, _DH:])
    o_ref[...] = jnp.concatenate([ca, cb], axis=1).astype(jnp.bfloat16)[None]


def _attn(q, k, v):
    qspec = pl.BlockSpec((1, _RQ, 2 * _DH), lambda b, h, i: (b, i, h))
    aspec = pl.BlockSpec(memory_space=pltpu.MemorySpace.HBM)
    return pl.pallas_call(
        _attn_body,
        grid=(_B, _H // 2, _TP // _RQ),
        in_specs=[qspec, aspec, aspec],
        out_specs=qspec,
        out_shape=jax.ShapeDtypeStruct((_B, _TP, _D), jnp.bfloat16),
        scratch_shapes=[
            pltpu.VMEM((_TP, 2 * _DH), jnp.bfloat16),
            pltpu.VMEM((_TP, 2 * _DH), jnp.bfloat16),
            pltpu.SemaphoreType.DMA,
            pltpu.SemaphoreType.DMA,
        ],
    )(q, k, v)


def _ffn_body(x_ref, ctx_ref, wo_hbm, w1_hbm, w2_hbm, h3_ref,
              wo_buf, w1_buf, w2_buf, sem):
    copies = [
        pltpu.make_async_copy(wo_hbm, wo_buf, sem.at[0]),
        pltpu.make_async_copy(w1_hbm, w1_buf, sem.at[1]),
        pltpu.make_async_copy(w2_hbm, w2_buf, sem.at[2]),
    ]

    @pl.when(pl.program_id(0) == 0)
    def _():
        for c in copies:
            c.start()
        for c in copies:
            c.wait()

    x2 = x_ref[...] + jnp.dot(ctx_ref[...], wo_buf[...],
                              preferred_element_type=jnp.float32)
    h2 = _ln(x2).astype(jnp.bfloat16)
    up = jnp.maximum(
        jnp.dot(h2, w1_buf[...], preferred_element_type=jnp.float32), 0.0
    ).astype(jnp.bfloat16)
    x3 = x2 + jnp.dot(up, w2_buf[...], preferred_element_type=jnp.float32)
    h3_ref[...] = _ln(x3).astype(jnp.bfloat16)


def _ffn(x, ctx, wo, w1, w2):
    row = pl.BlockSpec((_RB, _D), lambda i: (i, 0))
    aspec = pl.BlockSpec(memory_space=pltpu.MemorySpace.HBM)
    return pl.pallas_call(
        _ffn_body,
        grid=(_N // _RB,),
        in_specs=[row, row, aspec, aspec, aspec],
        out_specs=row,
        out_shape=jax.ShapeDtypeStruct((_N, _D), jnp.bfloat16),
        scratch_shapes=[
            pltpu.VMEM((_D, _D), jnp.bfloat16),
            pltpu.VMEM((_D, _FF), jnp.bfloat16),
            pltpu.VMEM((_FF, _D), jnp.bfloat16),
            pltpu.SemaphoreType.DMA((3,)),
        ],
    )(x, ctx, wo, w1, w2)


def _lm_body(h3_ref, lm_hbm, o_ref, lm_buf, sem):
    c = pltpu.make_async_copy(lm_hbm, lm_buf, sem)

    @pl.when(jnp.logical_and(pl.program_id(0) == 0, pl.program_id(1) == 0))
    def _():
        c.start()
        c.wait()

    o_ref[...] = jnp.dot(h3_ref[0], lm_buf[...],
                         preferred_element_type=jnp.float32)[None]


def _lm(h3, lm):
    nq = _TP // _RQ
    return pl.pallas_call(
        _lm_body,
        grid=(_B, nq),
        in_specs=[
            pl.BlockSpec((1, _RQ, _D), lambda b, i: (b, i, 0)),
            pl.BlockSpec(memory_space=pltpu.MemorySpace.HBM),
        ],
        out_specs=pl.BlockSpec((1, _RQ, _V), lambda b, i: (b, i, 0)),
        out_shape=jax.ShapeDtypeStruct((_B, _T, _V), jnp.float32),
        scratch_shapes=[
            pltpu.VMEM((_D, _V), jnp.bfloat16),
            pltpu.SemaphoreType.DMA,
        ],
    )(h3, lm)


def kernel(input_ids, attention_mask, prompt, emb_table, Wq, Wk, Wv, Wo,
           W1, W2, lm_head):
    del attention_mask
    gathered = _sc_gather(emb_table, input_ids.reshape(-1))
    x = jnp.concatenate(
        [
            jnp.broadcast_to(prompt[None], (_B, _P, _D)),
            gathered.reshape(_B, _S, _D),
            jnp.zeros((_B, _TP - _T, _D), jnp.float32),
        ],
        axis=1,
    )
    xf = x.reshape(_N, _D)
    bf = jnp.bfloat16
    q, k, v = _qkv(xf, Wq.astype(bf), Wk.astype(bf), Wv.astype(bf))
    ctx = _attn(q.reshape(_B, _TP, _D), k.reshape(_B, _TP, _D),
                v.reshape(_B, _TP, _D))
    h3 = _ffn(xf, ctx.reshape(_N, _D), Wo.astype(bf), W1.astype(bf),
              W2.astype(bf))
    return _lm(h3.reshape(_B, _TP, _D), lm_head.astype(bf))


# speedup vs baseline: 1.0322x; 1.0322x over previous
import jax
import jax.numpy as jnp
from jax.experimental import pallas as pl
from jax.experimental.pallas import tpu as pltpu
from jax.experimental.pallas import tpu_sc as plsc

_B, _S, _P, _D, _H, _V, _FF = 2, 2048, 20, 1024, 16, 8192, 4096
_DH = _D // _H
_T = _P + _S
_TP = 2176
_N = _B * _TP
_RB = 256
_RQ = 128
_EPS = 1e-5
_NEG = -1e30


def _ln(x):
    mu = jnp.mean(x, axis=-1, keepdims=True)
    var = jnp.mean(jnp.square(x - mu), axis=-1, keepdims=True)
    return (x - mu) * jax.lax.rsqrt(var + _EPS)


def _sc_gather(emb, ids_flat):
    n = ids_flat.shape[0]
    v, d = emb.shape
    c = 4
    dc = d // c
    w = 128
    nc = n * c
    idx = (ids_flat[:, None] * c
           + jnp.arange(c, dtype=jnp.int32)[None, :]).reshape(1, nc)
    embc = emb.reshape(v * c, dc)
    mesh = plsc.VectorSubcoreMesh(core_axis_name="c", subcore_axis_name="s")

    @pl.kernel(out_type=jax.ShapeDtypeStruct((nc, dc), emb.dtype), mesh=mesh)
    def k(emb_hbm, i_hbm, o_hbm):
        def body(i_vmem, o_vmem):
            pltpu.sync_copy(emb_hbm.at[i_vmem.at[0]], o_vmem)

        pltpu.emit_pipeline(
            body,
            grid=(nc // w,),
            in_specs=[pl.BlockSpec((1, w), index_map=lambda i: (0, i))],
            out_specs=[pl.BlockSpec((w, dc), index_map=lambda i: (i, 0))],
            core_axis_name=("c", "s"),
            dimension_semantics=(pltpu.PARALLEL,),
        )(i_hbm, o_hbm)

    return k(embc, idx).reshape(n, d)


def _qkv_body(x_ref, wq_ref, wk_ref, wv_ref, q_ref, k_ref, v_ref):
    h = _ln(x_ref[...]).astype(jnp.bfloat16)
    q_ref[...] = jnp.dot(h, wq_ref[...],
                         preferred_element_type=jnp.float32).astype(jnp.bfloat16)
    k_ref[...] = jnp.dot(h, wk_ref[...],
                         preferred_element_type=jnp.float32).astype(jnp.bfloat16)
    v_ref[...] = jnp.dot(h, wv_ref[...],
                         preferred_element_type=jnp.float32).astype(jnp.bfloat16)


def _qkv(x, wq, wk, wv):
    row = pl.BlockSpec((_RB, _D), lambda i: (i, 0))
    wsp = pl.BlockSpec((_D, _D), lambda i: (0, 0))
    out = jax.ShapeDtypeStruct((_N, _D), jnp.bfloat16)
    return pl.pallas_call(
        _qkv_body,
        grid=(_N // _RB,),
        in_specs=[row, wsp, wsp, wsp],
        out_specs=[row, row, row],
        out_shape=[out, out, out],
    )(x, wq, wk, wv)


def _attn_one(q, k, v):
    s = jax.lax.dot_general(q, k, (((1,), (1,)), ((), ())),
                            preferred_element_type=jnp.float32)
    s = s * (1.0 / 8.0)
    col = jax.lax.broadcasted_iota(jnp.int32, s.shape, 1)
    s = jnp.where(col < _T, s, _NEG)
    m = jnp.max(s, axis=-1, keepdims=True)
    e = jnp.exp(s - m)
    p = (e / jnp.sum(e, axis=-1, keepdims=True)).astype(jnp.bfloat16)
    return jnp.dot(p, v, preferred_element_type=jnp.float32)


def _attn_body(q_hbm, k_hbm, v_hbm, o_hbm, qb, kb, vb, ob, sems):
    b = pl.program_id(0)
    hp = pl.program_id(1)
    r0 = b * _TP
    c0 = hp * 2 * _DH
    cq = pltpu.make_async_copy(
        q_hbm.at[pl.ds(r0, _TP), pl.ds(c0, 2 * _DH)], qb, sems.at[0])
    ck = pltpu.make_async_copy(
        k_hbm.at[pl.ds(r0, _TP), pl.ds(c0, 2 * _DH)], kb, sems.at[1])
    cv = pltpu.make_async_copy(
        v_hbm.at[pl.ds(r0, _TP), pl.ds(c0, 2 * _DH)], vb, sems.at[2])
    cq.start()
    ck.start()
    cv.start()
    cq.wait()
    ck.wait()
    cv.wait()
    k = kb[...]
    v = vb[...]
    ka, kb2 = k[:, :_DH], k[:, _DH:]
    va, vb2 = v[:, :_DH], v[:, _DH:]
    for ti in range(_TP // _RQ):
        q = qb[pl.ds(ti * _RQ, _RQ), :]
        ca = _attn_one(q[:, :_DH], ka, va)
        cb_ = _attn_one(q[:, _DH:], kb2, vb2)
        ob[pl.ds(ti * _RQ, _RQ), :] = jnp.concatenate(
            [ca, cb_], axis=1).astype(jnp.bfloat16)
    co = pltpu.make_async_copy(
        ob, o_hbm.at[pl.ds(r0, _TP), pl.ds(c0, 2 * _DH)], sems.at[3])
    co.start()
    co.wait()


def _attn(q, k, v):
    aspec = pl.BlockSpec(memory_space=pltpu.MemorySpace.HBM)
    buf = pltpu.VMEM((_TP, 2 * _DH), jnp.bfloat16)
    return pl.pallas_call(
        _attn_body,
        grid=(_B, _H // 2),
        in_specs=[aspec, aspec, aspec],
        out_specs=aspec,
        out_shape=jax.ShapeDtypeStruct((_N, _D), jnp.bfloat16),
        scratch_shapes=[buf, buf, buf, buf, pltpu.SemaphoreType.DMA((4,))],
    )(q, k, v)


def _ffn_body(x_ref, ctx_ref, wo_hbm, w1_hbm, w2_hbm, h3_ref,
              wo_buf, w1_buf, w2_buf, sem):
    copies = [
        pltpu.make_async_copy(wo_hbm, wo_buf, sem.at[0]),
        pltpu.make_async_copy(w1_hbm, w1_buf, sem.at[1]),
        pltpu.make_async_copy(w2_hbm, w2_buf, sem.at[2]),
    ]

    @pl.when(pl.program_id(0) == 0)
    def _():
        for c in copies:
            c.start()
        for c in copies:
            c.wait()

    x2 = x_ref[...] + jnp.dot(ctx_ref[...], wo_buf[...],
                              preferred_element_type=jnp.float32)
    h2 = _ln(x2).astype(jnp.bfloat16)
    up = jnp.maximum(
        jnp.dot(h2, w1_buf[...], preferred_element_type=jnp.float32), 0.0
    ).astype(jnp.bfloat16)
    x3 = x2 + jnp.dot(up, w2_buf[...], preferred_element_type=jnp.float32)
    h3_ref[...] = _ln(x3).astype(jnp.bfloat16)


def _ffn(x, ctx, wo, w1, w2):
    row = pl.BlockSpec((_RB, _D), lambda i: (i, 0))
    aspec = pl.BlockSpec(memory_space=pltpu.MemorySpace.HBM)
    return pl.pallas_call(
        _ffn_body,
        grid=(_N // _RB,),
        in_specs=[row, row, aspec, aspec, aspec],
        out_specs=row,
        out_shape=jax.ShapeDtypeStruct((_N, _D), jnp.bfloat16),
        scratch_shapes=[
            pltpu.VMEM((_D, _D), jnp.bfloat16),
            pltpu.VMEM((_D, _FF), jnp.bfloat16),
            pltpu.VMEM((_FF, _D), jnp.bfloat16),
            pltpu.SemaphoreType.DMA((3,)),
        ],
    )(x, ctx, wo, w1, w2)


_VB = 512


def _lm_body(h3_ref, lm_ref, o_ref):
    lg = jnp.dot(h3_ref[0], lm_ref[...], preferred_element_type=jnp.float32)
    o_ref[...] = lg[:_T][None]


def _lm(h3, lm):
    return pl.pallas_call(
        _lm_body,
        grid=(_B, _V // _VB),
        in_specs=[
            pl.BlockSpec((1, _TP, _D), lambda b, vb: (b, 0, 0)),
            pl.BlockSpec((_D, _VB), lambda b, vb: (0, vb)),
        ],
        out_specs=pl.BlockSpec((1, _T, _VB), lambda b, vb: (b, 0, vb)),
        out_shape=jax.ShapeDtypeStruct((_B, _T, _V), jnp.float32),
    )(h3.reshape(_B, _TP, _D), lm)


def kernel(input_ids, attention_mask, prompt, emb_table, Wq, Wk, Wv, Wo,
           W1, W2, lm_head):
    del attention_mask
    gathered = _sc_gather(emb_table, input_ids.reshape(-1))
    zpad = jnp.zeros((_TP - _T, _D), jnp.float32)
    xf = jnp.concatenate(
        [prompt, gathered[:_S], zpad, prompt, gathered[_S:], zpad], axis=0)
    bf = jnp.bfloat16
    q, k, v = _qkv(xf, Wq.astype(bf), Wk.astype(bf), Wv.astype(bf))
    ctx = _attn(q, k, v)
    h3 = _ffn(xf, ctx, Wo.astype(bf), W1.astype(bf), W2.astype(bf))
    return _lm(h3, lm_head.astype(bf))


# speedup vs baseline: 1.3321x; 1.2906x over previous
import jax
import jax.numpy as jnp
from jax.experimental import pallas as pl
from jax.experimental.pallas import tpu as pltpu
from jax.experimental.pallas import tpu_sc as plsc

_B, _S, _P, _D, _H, _V, _FF = 2, 2048, 20, 1024, 16, 8192, 4096
_DH = _D // _H
_T = _P + _S
_TP = 2176
_N = _B * _TP
_RB = 256
_RQ = 128
_EPS = 1e-5
_NEG = -1e30


def _ln(x):
    mu = jnp.mean(x, axis=-1, keepdims=True)
    var = jnp.mean(jnp.square(x - mu), axis=-1, keepdims=True)
    return (x - mu) * jax.lax.rsqrt(var + _EPS)


def _sc_gather(emb, ids_flat):
    n = ids_flat.shape[0]
    v, d = emb.shape
    c = 4
    dc = d // c
    w = 128
    nc = n * c
    idx = (ids_flat[:, None] * c
           + jnp.arange(c, dtype=jnp.int32)[None, :]).reshape(1, nc)
    embc = emb.reshape(v * c, dc)
    mesh = plsc.VectorSubcoreMesh(core_axis_name="c", subcore_axis_name="s")

    @pl.kernel(out_type=jax.ShapeDtypeStruct((nc, dc), emb.dtype), mesh=mesh)
    def k(emb_hbm, i_hbm, o_hbm):
        def body(i_vmem, o_vmem):
            pltpu.sync_copy(emb_hbm.at[i_vmem.at[0]], o_vmem)

        pltpu.emit_pipeline(
            body,
            grid=(nc // w,),
            in_specs=[pl.BlockSpec((1, w), index_map=lambda i: (0, i))],
            out_specs=[pl.BlockSpec((w, dc), index_map=lambda i: (i, 0))],
            core_axis_name=("c", "s"),
            dimension_semantics=(pltpu.PARALLEL,),
        )(i_hbm, o_hbm)

    return k(embc, idx).reshape(n, d)


def _qkv_body(x_ref, wq_ref, wk_ref, wv_ref, q_ref, k_ref, v_ref):
    h = _ln(x_ref[...]).astype(jnp.bfloat16)
    q_ref[...] = jnp.dot(h, wq_ref[...],
                         preferred_element_type=jnp.float32).astype(jnp.bfloat16)
    k_ref[...] = jnp.dot(h, wk_ref[...],
                         preferred_element_type=jnp.float32).astype(jnp.bfloat16)
    v_ref[...] = jnp.dot(h, wv_ref[...],
                         preferred_element_type=jnp.float32).astype(jnp.bfloat16)


def _qkv(x, wq, wk, wv):
    row = pl.BlockSpec((_RB, _D), lambda i: (i, 0))
    wsp = pl.BlockSpec((_D, _D), lambda i: (0, 0))
    out = jax.ShapeDtypeStruct((_N, _D), jnp.bfloat16)
    return pl.pallas_call(
        _qkv_body,
        grid=(_N // _RB,),
        in_specs=[row, wsp, wsp, wsp],
        out_specs=[row, row, row],
        out_shape=[out, out, out],
    )(x, wq, wk, wv)


def _attn_body(q_hbm, k_hbm, v_hbm, o_hbm, qb, kb, vb, ob, sems):
    b = pl.program_id(0)
    hp = pl.program_id(1)
    r0 = b * _TP
    c0 = hp * 2 * _DH
    cq = pltpu.make_async_copy(
        q_hbm.at[pl.ds(r0, _TP), pl.ds(c0, 2 * _DH)], qb, sems.at[0])
    ck = pltpu.make_async_copy(
        k_hbm.at[pl.ds(r0, _TP), pl.ds(c0, 2 * _DH)], kb, sems.at[1])
    cv = pltpu.make_async_copy(
        v_hbm.at[pl.ds(r0, _TP), pl.ds(c0, 2 * _DH)], vb, sems.at[2])
    cq.start()
    ck.start()
    cv.start()
    cq.wait()
    ck.wait()
    cv.wait()
    qs = qb[...] * jnp.bfloat16(0.125)
    k = kb[...]
    v = vb[...]
    ones = jnp.ones((_TP, 1), jnp.bfloat16)
    heads = [
        (qs[:, :_DH], k[:, :_DH],
         jnp.concatenate([v[:, :_DH], ones], axis=1)),
        (qs[:, _DH:], k[:, _DH:],
         jnp.concatenate([v[:, _DH:], ones], axis=1)),
    ]
    npad = jnp.float32(_TP - _T)
    for ti in range(_TP // _RQ):
        outs = []
        for qh, kh, vh in heads:
            s = jax.lax.dot_general(
                qh[ti * _RQ:(ti + 1) * _RQ, :], kh, (((1,), (1,)), ((), ())),
                preferred_element_type=jnp.float32)
            e = jnp.exp(s).astype(jnp.bfloat16)
            cs = jnp.dot(e, vh, preferred_element_type=jnp.float32)
            outs.append(cs[:, :_DH] / (cs[:, _DH:] - npad))
        ob[pl.ds(ti * _RQ, _RQ), :] = jnp.concatenate(
            outs, axis=1).astype(jnp.bfloat16)
    co = pltpu.make_async_copy(
        ob, o_hbm.at[pl.ds(r0, _TP), pl.ds(c0, 2 * _DH)], sems.at[3])
    co.start()
    co.wait()


def _attn(q, k, v):
    aspec = pl.BlockSpec(memory_space=pltpu.MemorySpace.HBM)
    buf = pltpu.VMEM((_TP, 2 * _DH), jnp.bfloat16)
    return pl.pallas_call(
        _attn_body,
        grid=(_B, _H // 2),
        in_specs=[aspec, aspec, aspec],
        out_specs=aspec,
        out_shape=jax.ShapeDtypeStruct((_N, _D), jnp.bfloat16),
        scratch_shapes=[buf, buf, buf, buf, pltpu.SemaphoreType.DMA((4,))],
    )(q, k, v)


def _ffn_body(x_ref, ctx_ref, wo_hbm, w1_hbm, w2_hbm, h3_ref,
              wo_buf, w1_buf, w2_buf, sem):
    copies = [
        pltpu.make_async_copy(wo_hbm, wo_buf, sem.at[0]),
        pltpu.make_async_copy(w1_hbm, w1_buf, sem.at[1]),
        pltpu.make_async_copy(w2_hbm, w2_buf, sem.at[2]),
    ]

    @pl.when(pl.program_id(0) == 0)
    def _():
        for c in copies:
            c.start()
        for c in copies:
            c.wait()

    x2 = x_ref[...] + jnp.dot(ctx_ref[...], wo_buf[...],
                              preferred_element_type=jnp.float32)
    h2 = _ln(x2).astype(jnp.bfloat16)
    up = jnp.maximum(
        jnp.dot(h2, w1_buf[...], preferred_element_type=jnp.float32), 0.0
    ).astype(jnp.bfloat16)
    x3 = x2 + jnp.dot(up, w2_buf[...], preferred_element_type=jnp.float32)
    h3_ref[...] = _ln(x3).astype(jnp.bfloat16)


def _ffn(x, ctx, wo, w1, w2):
    row = pl.BlockSpec((_RB, _D), lambda i: (i, 0))
    aspec = pl.BlockSpec(memory_space=pltpu.MemorySpace.HBM)
    return pl.pallas_call(
        _ffn_body,
        grid=(_N // _RB,),
        in_specs=[row, row, aspec, aspec, aspec],
        out_specs=row,
        out_shape=jax.ShapeDtypeStruct((_N, _D), jnp.bfloat16),
        scratch_shapes=[
            pltpu.VMEM((_D, _D), jnp.bfloat16),
            pltpu.VMEM((_D, _FF), jnp.bfloat16),
            pltpu.VMEM((_FF, _D), jnp.bfloat16),
            pltpu.SemaphoreType.DMA((3,)),
        ],
    )(x, ctx, wo, w1, w2)


_VB = 512


def _lm_body(h3_ref, lm_ref, o_ref):
    lg = jnp.dot(h3_ref[0], lm_ref[...], preferred_element_type=jnp.float32)
    o_ref[...] = lg[:_T][None]


def _lm(h3, lm):
    return pl.pallas_call(
        _lm_body,
        grid=(_B, _V // _VB),
        in_specs=[
            pl.BlockSpec((1, _TP, _D), lambda b, vb: (b, 0, 0)),
            pl.BlockSpec((_D, _VB), lambda b, vb: (0, vb)),
        ],
        out_specs=pl.BlockSpec((1, _T, _VB), lambda b, vb: (b, 0, vb)),
        out_shape=jax.ShapeDtypeStruct((_B, _T, _V), jnp.float32),
    )(h3.reshape(_B, _TP, _D), lm)


def kernel(input_ids, attention_mask, prompt, emb_table, Wq, Wk, Wv, Wo,
           W1, W2, lm_head):
    del attention_mask
    gathered = _sc_gather(emb_table, input_ids.reshape(-1))
    zpad = jnp.zeros((_TP - _T, _D), jnp.float32)
    xf = jnp.concatenate(
        [prompt, gathered[:_S], zpad, prompt, gathered[_S:], zpad], axis=0)
    bf = jnp.bfloat16
    q, k, v = _qkv(xf, Wq.astype(bf), Wk.astype(bf), Wv.astype(bf))
    ctx = _attn(q, k, v)
    h3 = _ffn(xf, ctx, Wo.astype(bf), W1.astype(bf), W2.astype(bf))
    return _lm(h3, lm_head.astype(bf))


# speedup vs baseline: 1.3849x; 1.0396x over previous
import jax
import jax.numpy as jnp
from jax.experimental import pallas as pl
from jax.experimental.pallas import tpu as pltpu
from jax.experimental.pallas import tpu_sc as plsc

_B, _S, _P, _D, _H, _V, _FF = 2, 2048, 20, 1024, 16, 8192, 4096
_DH = _D // _H
_T = _P + _S
_TP = 2176
_N = _B * _TP
_RB = 256
_RQ = 128
_EPS = 1e-5
_NEG = -1e30


def _ln(x):
    mu = jnp.mean(x, axis=-1, keepdims=True)
    var = jnp.mean(jnp.square(x - mu), axis=-1, keepdims=True)
    return (x - mu) * jax.lax.rsqrt(var + _EPS)


def _sc_build_x(emb, ids_flat, prompt):
    c = 4
    dc = _D // c
    w = 128
    n = ids_flat.shape[0]
    nc = n * c
    j = jnp.arange(c, dtype=jnp.int32)[None, :]
    src_main = (ids_flat[:, None] * c + j).reshape(1, nc)
    tok = jnp.arange(n, dtype=jnp.int32)
    dstrow = (tok // _S) * _TP + _P + tok % _S
    dst_main = (dstrow[:, None] * c + j).reshape(1, nc)

    combo = jnp.concatenate(
        [prompt.reshape(_P * c, dc), jnp.zeros((1, dc), jnp.float32)], axis=0)
    pr = jnp.arange(_P, dtype=jnp.int32)
    zr = jnp.arange(_TP - _T, dtype=jnp.int32)
    src_tail, dst_tail = [], []
    for b in range(_B):
        src_tail.append((pr[:, None] * c + j).reshape(-1))
        dst_tail.append(((b * _TP + pr)[:, None] * c + j).reshape(-1))
        src_tail.append(jnp.full(((_TP - _T) * c,), _P * c, jnp.int32))
        dst_tail.append(((b * _TP + _T + zr)[:, None] * c + j).reshape(-1))
    src_tail = jnp.concatenate(src_tail).reshape(1, -1)
    dst_tail = jnp.concatenate(dst_tail).reshape(1, -1)
    ntail = src_tail.shape[1]

    embc = emb.reshape(-1, dc)
    mesh = plsc.VectorSubcoreMesh(core_axis_name="c", subcore_axis_name="s")

    @pl.kernel(out_type=jax.ShapeDtypeStruct((_N * c, dc), jnp.float32),
               mesh=mesh,
               scratch_types=[pltpu.VMEM((w, dc), jnp.float32)])
    def k(emb_hbm, combo_hbm, im_hbm, dm_hbm, it_hbm, dt_hbm, o_hbm, tmp):
        def body_main(i_v, d_v):
            pltpu.sync_copy(emb_hbm.at[i_v.at[0]], tmp)
            pltpu.sync_copy(tmp, o_hbm.at[d_v.at[0]])

        pltpu.emit_pipeline(
            body_main,
            grid=(nc // w,),
            in_specs=[pl.BlockSpec((1, w), index_map=lambda i: (0, i)),
                      pl.BlockSpec((1, w), index_map=lambda i: (0, i))],
            out_specs=[],
            core_axis_name=("c", "s"),
            dimension_semantics=(pltpu.PARALLEL,),
        )(im_hbm, dm_hbm)

        def body_tail(i_v, d_v):
            pltpu.sync_copy(combo_hbm.at[i_v.at[0]], tmp)
            pltpu.sync_copy(tmp, o_hbm.at[d_v.at[0]])

        pltpu.emit_pipeline(
            body_tail,
            grid=(ntail // w,),
            in_specs=[pl.BlockSpec((1, w), index_map=lambda i: (0, i)),
                      pl.BlockSpec((1, w), index_map=lambda i: (0, i))],
            out_specs=[],
            core_axis_name=("c", "s"),
            dimension_semantics=(pltpu.PARALLEL,),
        )(it_hbm, dt_hbm)

    return k(embc, combo, src_main, dst_main, src_tail, dst_tail)


def _qkv_body(x_ref, wq_ref, wk_ref, wv_ref, q_ref, k_ref, v_ref):
    h = _ln(x_ref[...]).astype(jnp.bfloat16)
    q_ref[...] = jnp.dot(h, wq_ref[...],
                         preferred_element_type=jnp.float32).astype(jnp.bfloat16)
    k_ref[...] = jnp.dot(h, wk_ref[...],
                         preferred_element_type=jnp.float32).astype(jnp.bfloat16)
    v_ref[...] = jnp.dot(h, wv_ref[...],
                         preferred_element_type=jnp.float32).astype(jnp.bfloat16)


def _qkv(x, wq, wk, wv):
    row = pl.BlockSpec((_RB, _D), lambda i: (i, 0))
    wsp = pl.BlockSpec((_D, _D), lambda i: (0, 0))
    out = jax.ShapeDtypeStruct((_N, _D), jnp.bfloat16)
    return pl.pallas_call(
        _qkv_body,
        grid=(_N // _RB,),
        in_specs=[row, wsp, wsp, wsp],
        out_specs=[row, row, row],
        out_shape=[out, out, out],
    )(x, wq, wk, wv)


def _attn_body(q_hbm, k_hbm, v_hbm, o_hbm, qb, kb, vb, ob, sems):
    b = pl.program_id(0)
    hp = pl.program_id(1)
    r0 = b * _TP
    c0 = hp * 2 * _DH
    cq = pltpu.make_async_copy(
        q_hbm.at[pl.ds(r0, _TP), pl.ds(c0, 2 * _DH)], qb, sems.at[0])
    ck = pltpu.make_async_copy(
        k_hbm.at[pl.ds(r0, _TP), pl.ds(c0, 2 * _DH)], kb, sems.at[1])
    cv = pltpu.make_async_copy(
        v_hbm.at[pl.ds(r0, _TP), pl.ds(c0, 2 * _DH)], vb, sems.at[2])
    cq.start()
    ck.start()
    cv.start()
    cq.wait()
    ck.wait()
    cv.wait()
    qs = qb[...] * jnp.bfloat16(0.125)
    k = kb[...]
    v = vb[...]
    ones = jnp.ones((_TP, 1), jnp.bfloat16)
    heads = [
        (qs[:, :_DH], k[:, :_DH],
         jnp.concatenate([v[:, :_DH], ones], axis=1)),
        (qs[:, _DH:], k[:, _DH:],
         jnp.concatenate([v[:, _DH:], ones], axis=1)),
    ]
    npad = jnp.float32(_TP - _T)
    for ti in range(_TP // _RQ):
        outs = []
        for qh, kh, vh in heads:
            s = jax.lax.dot_general(
                qh[ti * _RQ:(ti + 1) * _RQ, :], kh, (((1,), (1,)), ((), ())),
                preferred_element_type=jnp.float32)
            e = jnp.exp(s).astype(jnp.bfloat16)
            cs = jnp.dot(e, vh, preferred_element_type=jnp.float32)
            outs.append(cs[:, :_DH] / (cs[:, _DH:] - npad))
        ob[pl.ds(ti * _RQ, _RQ), :] = jnp.concatenate(
            outs, axis=1).astype(jnp.bfloat16)
    co = pltpu.make_async_copy(
        ob, o_hbm.at[pl.ds(r0, _TP), pl.ds(c0, 2 * _DH)], sems.at[3])
    co.start()
    co.wait()


def _attn(q, k, v):
    aspec = pl.BlockSpec(memory_space=pltpu.MemorySpace.HBM)
    buf = pltpu.VMEM((_TP, 2 * _DH), jnp.bfloat16)
    return pl.pallas_call(
        _attn_body,
        grid=(_B, _H // 2),
        in_specs=[aspec, aspec, aspec],
        out_specs=aspec,
        out_shape=jax.ShapeDtypeStruct((_N, _D), jnp.bfloat16),
        scratch_shapes=[buf, buf, buf, buf, pltpu.SemaphoreType.DMA((4,))],
    )(q, k, v)


def _ffn_body(x_ref, ctx_ref, wo_hbm, w1_hbm, w2_hbm, h3_ref,
              wo_buf, w1_buf, w2_buf, sem):
    copies = [
        pltpu.make_async_copy(wo_hbm, wo_buf, sem.at[0]),
        pltpu.make_async_copy(w1_hbm, w1_buf, sem.at[1]),
        pltpu.make_async_copy(w2_hbm, w2_buf, sem.at[2]),
    ]

    @pl.when(pl.program_id(0) == 0)
    def _():
        for c in copies:
            c.start()
        for c in copies:
            c.wait()

    x2 = x_ref[...] + jnp.dot(ctx_ref[...], wo_buf[...],
                              preferred_element_type=jnp.float32)
    h2 = _ln(x2).astype(jnp.bfloat16)
    up = jnp.maximum(
        jnp.dot(h2, w1_buf[...], preferred_element_type=jnp.float32), 0.0
    ).astype(jnp.bfloat16)
    x3 = x2 + jnp.dot(up, w2_buf[...], preferred_element_type=jnp.float32)
    h3_ref[...] = _ln(x3).astype(jnp.bfloat16)


def _ffn(x, ctx, wo, w1, w2):
    row = pl.BlockSpec((_RB, _D), lambda i: (i, 0))
    aspec = pl.BlockSpec(memory_space=pltpu.MemorySpace.HBM)
    return pl.pallas_call(
        _ffn_body,
        grid=(_N // _RB,),
        in_specs=[row, row, aspec, aspec, aspec],
        out_specs=row,
        out_shape=jax.ShapeDtypeStruct((_N, _D), jnp.bfloat16),
        scratch_shapes=[
            pltpu.VMEM((_D, _D), jnp.bfloat16),
            pltpu.VMEM((_D, _FF), jnp.bfloat16),
            pltpu.VMEM((_FF, _D), jnp.bfloat16),
            pltpu.SemaphoreType.DMA((3,)),
        ],
    )(x, ctx, wo, w1, w2)


_VB = 512


def _lm_body(h3_ref, lm_ref, o_ref):
    lg = jnp.dot(h3_ref[0], lm_ref[...], preferred_element_type=jnp.float32)
    o_ref[...] = lg[:_T][None]


def _lm(h3, lm):
    return pl.pallas_call(
        _lm_body,
        grid=(_B, _V // _VB),
        in_specs=[
            pl.BlockSpec((1, _TP, _D), lambda b, vb: (b, 0, 0)),
            pl.BlockSpec((_D, _VB), lambda b, vb: (0, vb)),
        ],
        out_specs=pl.BlockSpec((1, _T, _VB), lambda b, vb: (b, 0, vb)),
        out_shape=jax.ShapeDtypeStruct((_B, _T, _V), jnp.float32),
    )(h3.reshape(_B, _TP, _D), lm)


def kernel(input_ids, attention_mask, prompt, emb_table, Wq, Wk, Wv, Wo,
           W1, W2, lm_head):
    del attention_mask
    xf = _sc_build_x(emb_table, input_ids.reshape(-1),
                     prompt).reshape(_N, _D)
    bf = jnp.bfloat16
    q, k, v = _qkv(xf, Wq.astype(bf), Wk.astype(bf), Wv.astype(bf))
    ctx = _attn(q, k, v)
    h3 = _ffn(xf, ctx, Wo.astype(bf), W1.astype(bf), W2.astype(bf))
    return _lm(h3, lm_head.astype(bf))


# speedup vs baseline: 1.3971x; 1.0088x over previous
import jax
import jax.numpy as jnp
from jax.experimental import pallas as pl
from jax.experimental.pallas import tpu as pltpu
from jax.experimental.pallas import tpu_sc as plsc

_B, _S, _P, _D, _H, _V, _FF = 2, 2048, 20, 1024, 16, 8192, 4096
_DH = _D // _H
_T = _P + _S
_TP = 2176
_N = _B * _TP
_RB = 544
_RQ = 128
_EPS = 1e-5
_NEG = -1e30


def _ln(x):
    mu = jnp.mean(x, axis=-1, keepdims=True)
    var = jnp.mean(jnp.square(x - mu), axis=-1, keepdims=True)
    return (x - mu) * jax.lax.rsqrt(var + _EPS)


def _sc_build_x(emb, ids_flat, prompt):
    c = 4
    dc = _D // c
    w = 128
    n = ids_flat.shape[0]
    nc = n * c
    j = jnp.arange(c, dtype=jnp.int32)[None, :]
    src_main = (ids_flat[:, None] * c + j).reshape(1, nc)
    tok = jnp.arange(n, dtype=jnp.int32)
    dstrow = (tok // _S) * _TP + _P + tok % _S
    dst_main = (dstrow[:, None] * c + j).reshape(1, nc)

    combo = jnp.concatenate(
        [prompt.reshape(_P * c, dc), jnp.zeros((1, dc), jnp.float32)], axis=0)
    pr = jnp.arange(_P, dtype=jnp.int32)
    zr = jnp.arange(_TP - _T, dtype=jnp.int32)
    src_tail, dst_tail = [], []
    for b in range(_B):
        src_tail.append((pr[:, None] * c + j).reshape(-1))
        dst_tail.append(((b * _TP + pr)[:, None] * c + j).reshape(-1))
        src_tail.append(jnp.full(((_TP - _T) * c,), _P * c, jnp.int32))
        dst_tail.append(((b * _TP + _T + zr)[:, None] * c + j).reshape(-1))
    src_tail = jnp.concatenate(src_tail).reshape(1, -1)
    dst_tail = jnp.concatenate(dst_tail).reshape(1, -1)
    ntail = src_tail.shape[1]

    embc = emb.reshape(-1, dc)
    mesh = plsc.VectorSubcoreMesh(core_axis_name="c", subcore_axis_name="s")

    h = w // 2

    @pl.kernel(out_type=jax.ShapeDtypeStruct((_N * c, dc), jnp.float32),
               mesh=mesh,
               scratch_types=[pltpu.VMEM((h, dc), jnp.float32),
                              pltpu.VMEM((h, dc), jnp.float32),
                              pltpu.SemaphoreType.DMA((4,))])
    def k(emb_hbm, combo_hbm, im_hbm, dm_hbm, it_hbm, dt_hbm, o_hbm,
          t0, t1, sems):
        def make_body(src_hbm):
            def body(i_v, d_v):
                g0 = pltpu.make_async_copy(
                    src_hbm.at[i_v.at[0, pl.ds(0, h)]], t0, sems.at[0])
                g1 = pltpu.make_async_copy(
                    src_hbm.at[i_v.at[0, pl.ds(h, h)]], t1, sems.at[1])
                g0.start()
                g1.start()
                g0.wait()
                s0 = pltpu.make_async_copy(
                    t0, o_hbm.at[d_v.at[0, pl.ds(0, h)]], sems.at[2])
                s0.start()
                g1.wait()
                s1 = pltpu.make_async_copy(
                    t1, o_hbm.at[d_v.at[0, pl.ds(h, h)]], sems.at[3])
                s1.start()
                s0.wait()
                s1.wait()
            return body

        for src, isrc, idst, steps in (
                (emb_hbm, im_hbm, dm_hbm, nc // w),
                (combo_hbm, it_hbm, dt_hbm, ntail // w)):
            pltpu.emit_pipeline(
                make_body(src),
                grid=(steps,),
                in_specs=[pl.BlockSpec((1, w), index_map=lambda i: (0, i)),
                          pl.BlockSpec((1, w), index_map=lambda i: (0, i))],
                out_specs=[],
                core_axis_name=("c", "s"),
                dimension_semantics=(pltpu.PARALLEL,),
            )(isrc, idst)

    return k(embc, combo, src_main, dst_main, src_tail, dst_tail)


def _qkv_body(x_ref, wq_ref, wk_ref, wv_ref, q_ref, k_ref, v_ref):
    h = _ln(x_ref[...]).astype(jnp.bfloat16)
    q_ref[...] = jnp.dot(h, wq_ref[...],
                         preferred_element_type=jnp.float32).astype(jnp.bfloat16)
    k_ref[...] = jnp.dot(h, wk_ref[...],
                         preferred_element_type=jnp.float32).astype(jnp.bfloat16)
    v_ref[...] = jnp.dot(h, wv_ref[...],
                         preferred_element_type=jnp.float32).astype(jnp.bfloat16)


def _qkv(x, wq, wk, wv):
    row = pl.BlockSpec((_RB, _D), lambda i: (i, 0))
    wsp = pl.BlockSpec((_D, _D), lambda i: (0, 0))
    out = jax.ShapeDtypeStruct((_N, _D), jnp.bfloat16)
    return pl.pallas_call(
        _qkv_body,
        grid=(_N // _RB,),
        in_specs=[row, wsp, wsp, wsp],
        out_specs=[row, row, row],
        out_shape=[out, out, out],
    )(x, wq, wk, wv)


def _attn_body(q_hbm, k_hbm, v_hbm, o_hbm, qb, kb, vb, ob, sems):
    b = pl.program_id(0)
    hp = pl.program_id(1)
    r0 = b * _TP
    c0 = hp * 2 * _DH
    cq = pltpu.make_async_copy(
        q_hbm.at[pl.ds(r0, _TP), pl.ds(c0, 2 * _DH)], qb, sems.at[0])
    ck = pltpu.make_async_copy(
        k_hbm.at[pl.ds(r0, _TP), pl.ds(c0, 2 * _DH)], kb, sems.at[1])
    cv = pltpu.make_async_copy(
        v_hbm.at[pl.ds(r0, _TP), pl.ds(c0, 2 * _DH)], vb, sems.at[2])
    cq.start()
    ck.start()
    cv.start()
    cq.wait()
    ck.wait()
    cv.wait()
    qs = qb[...] * jnp.bfloat16(0.125)
    k = kb[...]
    v = vb[...]
    ones = jnp.ones((_TP, 1), jnp.bfloat16)
    heads = [
        (qs[:, :_DH], k[:, :_DH],
         jnp.concatenate([v[:, :_DH], ones], axis=1)),
        (qs[:, _DH:], k[:, _DH:],
         jnp.concatenate([v[:, _DH:], ones], axis=1)),
    ]
    npad = jnp.float32(_TP - _T)
    for ti in range(_TP // _RQ):
        outs = []
        for qh, kh, vh in heads:
            s = jax.lax.dot_general(
                qh[ti * _RQ:(ti + 1) * _RQ, :], kh, (((1,), (1,)), ((), ())),
                preferred_element_type=jnp.float32)
            e = jnp.exp(s).astype(jnp.bfloat16)
            cs = jnp.dot(e, vh, preferred_element_type=jnp.float32)
            outs.append(cs[:, :_DH] / (cs[:, _DH:] - npad))
        ob[pl.ds(ti * _RQ, _RQ), :] = jnp.concatenate(
            outs, axis=1).astype(jnp.bfloat16)
    co = pltpu.make_async_copy(
        ob, o_hbm.at[pl.ds(r0, _TP), pl.ds(c0, 2 * _DH)], sems.at[3])
    co.start()
    co.wait()


def _attn(q, k, v):
    aspec = pl.BlockSpec(memory_space=pltpu.MemorySpace.HBM)
    buf = pltpu.VMEM((_TP, 2 * _DH), jnp.bfloat16)
    return pl.pallas_call(
        _attn_body,
        grid=(_B, _H // 2),
        in_specs=[aspec, aspec, aspec],
        out_specs=aspec,
        out_shape=jax.ShapeDtypeStruct((_N, _D), jnp.bfloat16),
        scratch_shapes=[buf, buf, buf, buf, pltpu.SemaphoreType.DMA((4,))],
    )(q, k, v)


def _ffn_body(x_ref, ctx_ref, wo_hbm, w1_hbm, w2_hbm, h3_ref,
              wo_buf, w1_buf, w2_buf, sem):
    copies = [
        pltpu.make_async_copy(wo_hbm, wo_buf, sem.at[0]),
        pltpu.make_async_copy(w1_hbm, w1_buf, sem.at[1]),
        pltpu.make_async_copy(w2_hbm, w2_buf, sem.at[2]),
    ]

    @pl.when(pl.program_id(0) == 0)
    def _():
        for c in copies:
            c.start()
        for c in copies:
            c.wait()

    x2 = x_ref[...] + jnp.dot(ctx_ref[...], wo_buf[...],
                              preferred_element_type=jnp.float32)
    h2 = _ln(x2).astype(jnp.bfloat16)
    up = jnp.maximum(
        jnp.dot(h2, w1_buf[...], preferred_element_type=jnp.float32), 0.0
    ).astype(jnp.bfloat16)
    x3 = x2 + jnp.dot(up, w2_buf[...], preferred_element_type=jnp.float32)
    h3_ref[...] = _ln(x3).astype(jnp.bfloat16)


def _ffn(x, ctx, wo, w1, w2):
    row = pl.BlockSpec((_RB, _D), lambda i: (i, 0))
    aspec = pl.BlockSpec(memory_space=pltpu.MemorySpace.HBM)
    return pl.pallas_call(
        _ffn_body,
        grid=(_N // _RB,),
        in_specs=[row, row, aspec, aspec, aspec],
        out_specs=row,
        out_shape=jax.ShapeDtypeStruct((_N, _D), jnp.bfloat16),
        scratch_shapes=[
            pltpu.VMEM((_D, _D), jnp.bfloat16),
            pltpu.VMEM((_D, _FF), jnp.bfloat16),
            pltpu.VMEM((_FF, _D), jnp.bfloat16),
            pltpu.SemaphoreType.DMA((3,)),
        ],
    )(x, ctx, wo, w1, w2)


_VB = 512


def _lm_body(h3_ref, lm_ref, o_ref):
    lg = jnp.dot(h3_ref[0], lm_ref[...], preferred_element_type=jnp.float32)
    o_ref[...] = lg[:_T][None]


def _lm(h3, lm):
    return pl.pallas_call(
        _lm_body,
        grid=(_B, _V // _VB),
        in_specs=[
            pl.BlockSpec((1, _TP, _D), lambda b, vb: (b, 0, 0)),
            pl.BlockSpec((_D, _VB), lambda b, vb: (0, vb)),
        ],
        out_specs=pl.BlockSpec((1, _T, _VB), lambda b, vb: (b, 0, vb)),
        out_shape=jax.ShapeDtypeStruct((_B, _T, _V), jnp.float32),
    )(h3.reshape(_B, _TP, _D), lm)


def kernel(input_ids, attention_mask, prompt, emb_table, Wq, Wk, Wv, Wo,
           W1, W2, lm_head):
    del attention_mask
    xf = _sc_build_x(emb_table, input_ids.reshape(-1),
                     prompt).reshape(_N, _D)
    bf = jnp.bfloat16
    q, k, v = _qkv(xf, Wq.astype(bf), Wk.astype(bf), Wv.astype(bf))
    ctx = _attn(q, k, v)
    h3 = _ffn(xf, ctx, Wo.astype(bf), W1.astype(bf), W2.astype(bf))
    return _lm(h3, lm_head.astype(bf))
